# baseline (speedup 1.0000x reference)
import functools

import jax
import jax.numpy as jnp
from jax import lax
from jax.experimental import pallas as pl
from jax.experimental.pallas import tpu as pltpu
from jax.experimental.pallas import tpu_sc as plsc

_N_CLASSES = 1000000
_DIM = 64
_B = 16384

_NC = 2
_NS = 16
_NW = _NC * _NS
_BPW = _B // _NW
_L = 16

_mesh = plsc.VectorSubcoreMesh(core_axis_name="c", subcore_axis_name="s")


@functools.partial(
    pl.kernel,
    mesh=_mesh,
    out_type=jax.ShapeDtypeStruct((_B, _DIM), jnp.float32),
    compiler_params=pltpu.CompilerParams(use_tc_tiling_on_sc=False),
    scratch_types=[
        pltpu.VMEM((_BPW,), jnp.int32),
        pltpu.VMEM((_BPW,), jnp.int32),
        pltpu.VMEM((_BPW,), jnp.int32),
        pltpu.VMEM((_BPW, _DIM), jnp.float32),
        pltpu.SemaphoreType.DMA,
    ],
)
def _embed_lookup(cond_hbm, unc_hbm, table_hbm, out_hbm,
                  cond_v, unc_v, idx_v, rows_v, sem):
    wid = lax.axis_index("s") * _NC + lax.axis_index("c")
    base = wid * _BPW
    pltpu.sync_copy(cond_hbm.at[pl.ds(base, _BPW)], cond_v)
    pltpu.sync_copy(unc_hbm.at[pl.ds(base, _BPW)], unc_v)
    for i in range(_BPW // _L):
        sl = pl.ds(i * _L, _L)
        idx_v[sl] = jnp.where(unc_v[sl] != 0,
                              jnp.full((_L,), _N_CLASSES, jnp.int32),
                              cond_v[sl])
    pltpu.async_copy(table_hbm.at[idx_v], rows_v, sem).wait()
    pltpu.sync_copy(rows_v, out_hbm.at[pl.ds(base, _BPW)])


def kernel(condition, is_uncond, embed):
    return _embed_lookup(condition.astype(jnp.int32),
                         is_uncond.astype(jnp.int32),
                         embed)


# speedup vs baseline: 1.1376x; 1.1376x over previous
import functools

import jax
import jax.numpy as jnp
from jax import lax
from jax.experimental import pallas as pl
from jax.experimental.pallas import tpu as pltpu
from jax.experimental.pallas import tpu_sc as plsc

_N_CLASSES = 1000000
_DIM = 64
_B = 16384

_NC = 2
_NS = 16
_NW = _NC * _NS
_BPW = _B // _NW
_L = 16

_mesh = plsc.VectorSubcoreMesh(core_axis_name="c", subcore_axis_name="s")


@functools.partial(
    pl.kernel,
    mesh=_mesh,
    out_type=jax.ShapeDtypeStruct((_B, _DIM), jnp.float32),
    scratch_types=[
        pltpu.VMEM((_BPW,), jnp.int32),
        pltpu.VMEM((_BPW,), jnp.int32),
        pltpu.SemaphoreType.DMA,
    ],
)
def _embed_lookup(cond_hbm, unc_hbm, table_hbm, out_hbm, cond_v, unc_v, sem):
    wid = lax.axis_index("s") * _NC + lax.axis_index("c")
    base = wid * _BPW
    pltpu.sync_copy(cond_hbm.at[pl.ds(base, _BPW)], cond_v)
    pltpu.sync_copy(unc_hbm.at[pl.ds(base, _BPW)], unc_v)

    lanes = lax.iota(jnp.int32, _L)
    zeros = jnp.zeros((_L,), jnp.int32)

    def body(c, carry):
        cv = cond_v[pl.ds(c * _L, _L)]
        uv = unc_v[pl.ds(c * _L, _L)]
        idx16 = jnp.where(uv != 0, jnp.full((_L,), _N_CLASSES, jnp.int32), cv)
        for l in range(_L):
            idx = idx16[l]
            pltpu.async_copy(table_hbm.at[pl.ds(idx, 1)],
                             out_hbm.at[pl.ds(base + c * _L + l, 1)], sem)
        return carry

    lax.fori_loop(0, _BPW // _L, body, 0)
    pltpu.make_async_copy(out_hbm.at[pl.ds(base, _BPW)],
                          out_hbm.at[pl.ds(base, _BPW)], sem).wait()


def kernel(condition, is_uncond, embed):
    return _embed_lookup(condition.astype(jnp.int32),
                         is_uncond.astype(jnp.int32),
                         embed)


# speedup vs baseline: 1.1528x; 1.0133x over previous
import functools

import jax
import jax.numpy as jnp
from jax import lax
from jax.experimental import pallas as pl
from jax.experimental.pallas import tpu as pltpu
from jax.experimental.pallas import tpu_sc as plsc

_N_CLASSES = 1000000
_DIM = 64
_B = 16384

_NC = 2
_NS = 16
_NW = _NC * _NS
_BPW = _B // _NW
_L = 16

_mesh = plsc.VectorSubcoreMesh(core_axis_name="c", subcore_axis_name="s")


@functools.partial(
    pl.kernel,
    mesh=_mesh,
    out_type=jax.ShapeDtypeStruct((_B, _DIM), jnp.float32),
    scratch_types=[
        pltpu.VMEM((_BPW,), jnp.int32),
        pltpu.VMEM((_BPW,), jnp.int32),
        pltpu.VMEM((_BPW, _DIM), jnp.float32),
        pltpu.SemaphoreType.DMA,
    ],
)
def _embed_lookup(cond_hbm, unc_hbm, table_hbm, out_hbm,
                  cond_v, unc_v, rows_v, sem):
    wid = lax.axis_index("s") * _NC + lax.axis_index("c")
    base = wid * _BPW
    pltpu.sync_copy(cond_hbm.at[pl.ds(base, _BPW)], cond_v)
    pltpu.sync_copy(unc_hbm.at[pl.ds(base, _BPW)], unc_v)

    def body(c, carry):
        cv = cond_v[pl.ds(c * _L, _L)]
        uv = unc_v[pl.ds(c * _L, _L)]
        idx16 = jnp.where(uv != 0, jnp.full((_L,), _N_CLASSES, jnp.int32), cv)
        for l in range(_L):
            idx = idx16[l]
            pltpu.async_copy(table_hbm.at[pl.ds(idx, 1)],
                             rows_v.at[pl.ds(c * _L + l, 1)], sem)
        return carry

    lax.fori_loop(0, _BPW // _L, body, 0)
    pltpu.make_async_copy(table_hbm.at[pl.ds(0, _BPW)], rows_v, sem).wait()
    pltpu.sync_copy(rows_v, out_hbm.at[pl.ds(base, _BPW)])


def kernel(condition, is_uncond, embed):
    return _embed_lookup(condition.astype(jnp.int32),
                         is_uncond.astype(jnp.int32),
                         embed)
